# Initial kernel scaffold; baseline (speedup 1.0000x reference)
import functools

import jax
import jax.numpy as jnp
from jax import lax
from jax.experimental import pallas as pl
from jax.experimental.pallas import tpu as pltpu
from jax.experimental.pallas import tpu_sc as plsc

B = 16384
H = 50
D = 32
NC = 2
NS = 16
NW = NC * NS
BPW = B // NW


def _body(uid, ugen, iid, cid, hist_i, hist_c,
          emb_uid, emb_ugen, emb_iid, emb_cid,
          out, idx_v, hist_v, rows_v, sem):
    wid = lax.axis_index("s") * NC + lax.axis_index("c")
    base = wid * BPW

    for feat, table, col in ((uid, emb_uid, 0), (ugen, emb_ugen, D),
                             (iid, emb_iid, 2 * D), (cid, emb_cid, 3 * D)):
        pltpu.sync_copy(feat.at[pl.ds(base, BPW)], idx_v)
        pltpu.async_copy(table.at[idx_v], rows_v, sem).wait()
        pltpu.sync_copy(rows_v, out.at[pl.ds(base, BPW), pl.ds(col, D)])

    for hist, table, col in ((hist_i, emb_iid, 4 * D), (hist_c, emb_cid, 5 * D)):
        pltpu.sync_copy(hist.at[:, pl.ds(base, BPW)], hist_v)
        pltpu.async_copy(table.at[hist_v.at[0]], rows_v, sem).wait()

        def step(h, _):
            pltpu.async_copy(table.at[hist_v.at[h]], rows_v, sem, add=True).wait()
            return 0

        lax.fori_loop(1, H, step, 0)

        def scale(i, _):
            for c in (0, 16):
                rows_v[i, pl.ds(c, 16)] = rows_v[i, pl.ds(c, 16)] * (1.0 / H)
            return 0

        lax.fori_loop(0, BPW, scale, 0)
        pltpu.sync_copy(rows_v, out.at[pl.ds(base, BPW), pl.ds(col, D)])


@jax.jit
def _run(uid, ugen, iid, cid, hist_i_t, hist_c_t,
         emb_uid, emb_ugen, emb_iid, emb_cid):
    mesh = plsc.VectorSubcoreMesh(core_axis_name="c", subcore_axis_name="s",
                                  num_cores=NC, num_subcores=NS)
    f = pl.kernel(
        _body,
        out_type=jax.ShapeDtypeStruct((B, 6 * D), jnp.float32),
        mesh=mesh,
        scratch_types=[
            pltpu.VMEM((BPW,), jnp.int32),
            pltpu.VMEM((H, BPW), jnp.int32),
            pltpu.VMEM((BPW, D), jnp.float32),
            pltpu.SemaphoreType.DMA,
        ],
    )
    return f(uid, ugen, iid, cid, hist_i_t, hist_c_t,
             emb_uid, emb_ugen, emb_iid, emb_cid)


def kernel(user_id, user_gender, item_id, cate_id, hist_item_id, hist_cate_id,
           labels, emb_user_id, emb_user_gender, emb_item_id, emb_cate_id):
    uid = user_id.reshape(B).astype(jnp.int32)
    ugen = user_gender.reshape(B).astype(jnp.int32)
    iid = item_id.reshape(B).astype(jnp.int32)
    cid = cate_id.reshape(B).astype(jnp.int32)
    hist_i_t = hist_item_id.astype(jnp.int32).T
    hist_c_t = hist_cate_id.astype(jnp.int32).T
    return _run(uid, ugen, iid, cid, hist_i_t, hist_c_t,
                emb_user_id, emb_user_gender, emb_item_id, emb_cate_id)


# speedup vs baseline: 3.9283x; 3.9283x over previous
import functools

import jax
import jax.numpy as jnp
from jax import lax
from jax.experimental import pallas as pl
from jax.experimental.pallas import tpu as pltpu
from jax.experimental.pallas import tpu_sc as plsc

B = 16384
H = 50
D = 32
NC = 2
NS = 16
NW = NC * NS
BPW = B // NW


def _body(uid, ugen, iid, cid, hist_i, hist_c,
          emb_uid, emb_ugen, emb_iid, emb_cid,
          out_u, out_g, out_i, out_c, out_hi, out_hc,
          idx_v, hist_v, rows_v, sem):
    wid = lax.axis_index("s") * NC + lax.axis_index("c")
    base = wid * BPW

    for feat, table, out in ((uid, emb_uid, out_u), (ugen, emb_ugen, out_g),
                             (iid, emb_iid, out_i), (cid, emb_cid, out_c)):
        pltpu.sync_copy(feat.at[pl.ds(base, BPW)], idx_v)
        pltpu.async_copy(table.at[idx_v], rows_v, sem).wait()
        pltpu.sync_copy(rows_v, out.at[pl.ds(base, BPW), :])

    for hist, table, out in ((hist_i, emb_iid, out_hi), (hist_c, emb_cid, out_hc)):
        pltpu.sync_copy(hist.at[:, pl.ds(base, BPW)], hist_v)
        pltpu.async_copy(table.at[hist_v.at[0]], rows_v, sem).wait()

        def step(h, _):
            pltpu.async_copy(table.at[hist_v.at[h]], rows_v, sem, add=True).wait()
            return 0

        lax.fori_loop(1, H, step, 0)

        def scale(i, _):
            for c in (0, 16):
                rows_v[i, pl.ds(c, 16)] = rows_v[i, pl.ds(c, 16)] * (1.0 / H)
            return 0

        lax.fori_loop(0, BPW, scale, 0)
        pltpu.sync_copy(rows_v, out.at[pl.ds(base, BPW), :])


@jax.jit
def _run(uid, ugen, iid, cid, hist_i_t, hist_c_t,
         emb_uid, emb_ugen, emb_iid, emb_cid):
    mesh = plsc.VectorSubcoreMesh(core_axis_name="c", subcore_axis_name="s",
                                  num_cores=NC, num_subcores=NS)
    f = pl.kernel(
        _body,
        out_type=[jax.ShapeDtypeStruct((B, D), jnp.float32)] * 6,
        mesh=mesh,
        scratch_types=[
            pltpu.VMEM((BPW,), jnp.int32),
            pltpu.VMEM((H, BPW), jnp.int32),
            pltpu.VMEM((BPW, D), jnp.float32),
            pltpu.SemaphoreType.DMA,
        ],
        compiler_params=pltpu.CompilerParams(use_tc_tiling_on_sc=False),
    )
    outs = f(uid, ugen, iid, cid, hist_i_t, hist_c_t,
             emb_uid, emb_ugen, emb_iid, emb_cid)
    return jnp.concatenate(outs, axis=-1)


def kernel(user_id, user_gender, item_id, cate_id, hist_item_id, hist_cate_id,
           labels, emb_user_id, emb_user_gender, emb_item_id, emb_cate_id):
    uid = user_id.reshape(B).astype(jnp.int32)
    ugen = user_gender.reshape(B).astype(jnp.int32)
    iid = item_id.reshape(B).astype(jnp.int32)
    cid = cate_id.reshape(B).astype(jnp.int32)
    hist_i_t = hist_item_id.astype(jnp.int32).T
    hist_c_t = hist_cate_id.astype(jnp.int32).T
    return _run(uid, ugen, iid, cid, hist_i_t, hist_c_t,
                emb_user_id, emb_user_gender, emb_item_id, emb_cate_id)


# speedup vs baseline: 4.1530x; 1.0572x over previous
import functools

import jax
import jax.numpy as jnp
from jax import lax
from jax.experimental import pallas as pl
from jax.experimental.pallas import tpu as pltpu
from jax.experimental.pallas import tpu_sc as plsc

B = 16384
H = 50
D = 32
NC = 2
NS = 16
NW = NC * NS
BPW = B // NW


ACC_BYTES = BPW * D * 4


def _body(uid, ugen, iid, cid, hist_i, hist_c,
          emb_uid, emb_ugen, emb_iid, emb_cid,
          out_u, out_g, out_i, out_c, out_hi, out_hc,
          idx4, hiv, hcv, r0, r1, acc_i, acc_c,
          s_idx, s0, s1, si, sc):
    wid = lax.axis_index("s") * NC + lax.axis_index("c")
    base = wid * BPW
    bsl = pl.ds(base, BPW)

    d_idx = [pltpu.async_copy(feat.at[bsl], idx4.at[k], s_idx)
             for k, feat in enumerate((uid, ugen, iid, cid))]
    d_hi = pltpu.async_copy(hist_i.at[:, bsl], hiv, s_idx)
    d_hc = pltpu.async_copy(hist_c.at[:, bsl], hcv, s_idx)

    zeros = jnp.zeros((16,), jnp.float32)

    def zero(i, _):
        for acc in (acc_i, acc_c):
            for c in (0, 16):
                acc[i, pl.ds(c, 16)] = zeros
        return 0

    lax.fori_loop(0, BPW, zero, 0)

    for d in d_idx:
        d.wait()
    d_hi.wait()
    d_hc.wait()

    g0 = pltpu.async_copy(emb_uid.at[idx4.at[0]], r0, s0)
    g1 = pltpu.async_copy(emb_ugen.at[idx4.at[1]], r1, s1)
    g0.wait()
    w0 = pltpu.async_copy(r0, out_u.at[bsl, :], s0)
    g1.wait()
    w1 = pltpu.async_copy(r1, out_g.at[bsl, :], s1)
    w0.wait()
    g2 = pltpu.async_copy(emb_iid.at[idx4.at[2]], r0, s0)
    w1.wait()
    g3 = pltpu.async_copy(emb_cid.at[idx4.at[3]], r1, s1)

    def fire_i(h, _):
        pltpu.async_copy(emb_iid.at[hiv.at[h]], acc_i, si, add=True)
        return 0

    def fire_c(h, _):
        pltpu.async_copy(emb_cid.at[hcv.at[h]], acc_c, sc, add=True)
        return 0

    lax.fori_loop(0, H, fire_i, 0)
    lax.fori_loop(0, H, fire_c, 0)

    g2.wait()
    pltpu.sync_copy(r0, out_i.at[bsl, :])
    g3.wait()
    pltpu.sync_copy(r1, out_c.at[bsl, :])

    for acc, sem, out in ((acc_i, si, out_hi), (acc_c, sc, out_hc)):
        def drain(h, _):
            pltpu.make_async_copy(out_hi.at[bsl, :], acc, sem).wait()
            return 0

        lax.fori_loop(0, H, drain, 0)

        def scale(i, _):
            for c in (0, 16):
                acc[i, pl.ds(c, 16)] = acc[i, pl.ds(c, 16)] * (1.0 / H)
            return 0

        lax.fori_loop(0, BPW, scale, 0)
        pltpu.sync_copy(acc, out.at[bsl, :])


@jax.jit
def _run(uid, ugen, iid, cid, hist_i_t, hist_c_t,
         emb_uid, emb_ugen, emb_iid, emb_cid):
    mesh = plsc.VectorSubcoreMesh(core_axis_name="c", subcore_axis_name="s",
                                  num_cores=NC, num_subcores=NS)
    f = pl.kernel(
        _body,
        out_type=[jax.ShapeDtypeStruct((B, D), jnp.float32)] * 6,
        mesh=mesh,
        scratch_types=[
            pltpu.VMEM((4, BPW), jnp.int32),
            pltpu.VMEM((H, BPW), jnp.int32),
            pltpu.VMEM((H, BPW), jnp.int32),
            pltpu.VMEM((BPW, D), jnp.float32),
            pltpu.VMEM((BPW, D), jnp.float32),
            pltpu.VMEM((BPW, D), jnp.float32),
            pltpu.VMEM((BPW, D), jnp.float32),
            pltpu.SemaphoreType.DMA,
            pltpu.SemaphoreType.DMA,
            pltpu.SemaphoreType.DMA,
            pltpu.SemaphoreType.DMA,
            pltpu.SemaphoreType.DMA,
        ],
        compiler_params=pltpu.CompilerParams(use_tc_tiling_on_sc=False),
    )
    outs = f(uid, ugen, iid, cid, hist_i_t, hist_c_t,
             emb_uid, emb_ugen, emb_iid, emb_cid)
    return jnp.concatenate(outs, axis=-1)


def kernel(user_id, user_gender, item_id, cate_id, hist_item_id, hist_cate_id,
           labels, emb_user_id, emb_user_gender, emb_item_id, emb_cate_id):
    uid = user_id.reshape(B).astype(jnp.int32)
    ugen = user_gender.reshape(B).astype(jnp.int32)
    iid = item_id.reshape(B).astype(jnp.int32)
    cid = cate_id.reshape(B).astype(jnp.int32)
    hist_i_t = hist_item_id.astype(jnp.int32).T
    hist_c_t = hist_cate_id.astype(jnp.int32).T
    return _run(uid, ugen, iid, cid, hist_i_t, hist_c_t,
                emb_user_id, emb_user_gender, emb_item_id, emb_cate_id)


# speedup vs baseline: 4.1556x; 1.0006x over previous
import functools

import jax
import jax.numpy as jnp
from jax import lax
from jax.experimental import pallas as pl
from jax.experimental.pallas import tpu as pltpu
from jax.experimental.pallas import tpu_sc as plsc

B = 16384
H = 50
D = 32
NC = 2
NS = 16
NW = NC * NS
BPW = B // NW


ACC_BYTES = BPW * D * 4


def _body(uid, ugen, iid, cid, hist_i, hist_c,
          emb_uid, emb_ugen, emb_iid, emb_cid,
          out_u, out_g, out_i, out_c, out_hi, out_hc,
          idx4, hiv, hcv, r0, r1, acc_i, acc_c,
          s_idx, s0, s1, si, sc):
    wid = lax.axis_index("s") * NC + lax.axis_index("c")
    base = wid * BPW
    bsl = pl.ds(base, BPW)

    d_idx = [pltpu.async_copy(feat.at[bsl], idx4.at[k], s_idx)
             for k, feat in enumerate((uid, ugen, iid, cid))]

    def load_hist(h, _):
        pltpu.async_copy(hist_i.at[pl.ds(h * B + base, BPW)], hiv.at[h], s_idx)
        pltpu.async_copy(hist_c.at[pl.ds(h * B + base, BPW)], hcv.at[h], s_idx)
        return 0

    lax.fori_loop(0, H, load_hist, 0)

    zeros = jnp.zeros((16,), jnp.float32)

    def zero(i, _):
        for acc in (acc_i, acc_c):
            for c in (0, 16):
                acc[i, pl.ds(c, 16)] = zeros
        return 0

    lax.fori_loop(0, BPW, zero, 0)

    for d in d_idx:
        d.wait()

    def drain_hist(h, _):
        pltpu.make_async_copy(hist_i.at[pl.ds(base, BPW)], hiv.at[0], s_idx).wait()
        pltpu.make_async_copy(hist_c.at[pl.ds(base, BPW)], hcv.at[0], s_idx).wait()
        return 0

    lax.fori_loop(0, H, drain_hist, 0)

    g0 = pltpu.async_copy(emb_uid.at[idx4.at[0]], r0, s0)
    g1 = pltpu.async_copy(emb_ugen.at[idx4.at[1]], r1, s1)
    g0.wait()
    w0 = pltpu.async_copy(r0, out_u.at[bsl, :], s0)
    g1.wait()
    w1 = pltpu.async_copy(r1, out_g.at[bsl, :], s1)
    w0.wait()
    g2 = pltpu.async_copy(emb_iid.at[idx4.at[2]], r0, s0)
    w1.wait()
    g3 = pltpu.async_copy(emb_cid.at[idx4.at[3]], r1, s1)

    def fire_i(h, _):
        pltpu.async_copy(emb_iid.at[hiv.at[h]], acc_i, si, add=True)
        return 0

    def fire_c(h, _):
        pltpu.async_copy(emb_cid.at[hcv.at[h]], acc_c, sc, add=True)
        return 0

    lax.fori_loop(0, H, fire_i, 0)
    lax.fori_loop(0, H, fire_c, 0)

    g2.wait()
    pltpu.sync_copy(r0, out_i.at[bsl, :])
    g3.wait()
    pltpu.sync_copy(r1, out_c.at[bsl, :])

    for acc, sem, out in ((acc_i, si, out_hi), (acc_c, sc, out_hc)):
        def drain(h, _):
            pltpu.make_async_copy(out_hi.at[bsl, :], acc, sem).wait()
            return 0

        lax.fori_loop(0, H, drain, 0)

        def scale(i, _):
            for c in (0, 16):
                acc[i, pl.ds(c, 16)] = acc[i, pl.ds(c, 16)] * (1.0 / H)
            return 0

        lax.fori_loop(0, BPW, scale, 0)
        pltpu.sync_copy(acc, out.at[bsl, :])


@jax.jit
def _run(uid, ugen, iid, cid, hist_i_t, hist_c_t,
         emb_uid, emb_ugen, emb_iid, emb_cid):
    mesh = plsc.VectorSubcoreMesh(core_axis_name="c", subcore_axis_name="s",
                                  num_cores=NC, num_subcores=NS)
    f = pl.kernel(
        _body,
        out_type=[jax.ShapeDtypeStruct((B, D), jnp.float32)] * 6,
        mesh=mesh,
        scratch_types=[
            pltpu.VMEM((4, BPW), jnp.int32),
            pltpu.VMEM((H, BPW), jnp.int32),
            pltpu.VMEM((H, BPW), jnp.int32),
            pltpu.VMEM((BPW, D), jnp.float32),
            pltpu.VMEM((BPW, D), jnp.float32),
            pltpu.VMEM((BPW, D), jnp.float32),
            pltpu.VMEM((BPW, D), jnp.float32),
            pltpu.SemaphoreType.DMA,
            pltpu.SemaphoreType.DMA,
            pltpu.SemaphoreType.DMA,
            pltpu.SemaphoreType.DMA,
            pltpu.SemaphoreType.DMA,
        ],
        compiler_params=pltpu.CompilerParams(use_tc_tiling_on_sc=False),
    )
    outs = f(uid, ugen, iid, cid, hist_i_t, hist_c_t,
             emb_uid, emb_ugen, emb_iid, emb_cid)
    return jnp.concatenate(outs, axis=-1)


def kernel(user_id, user_gender, item_id, cate_id, hist_item_id, hist_cate_id,
           labels, emb_user_id, emb_user_gender, emb_item_id, emb_cate_id):
    uid = user_id.reshape(B).astype(jnp.int32)
    ugen = user_gender.reshape(B).astype(jnp.int32)
    iid = item_id.reshape(B).astype(jnp.int32)
    cid = cate_id.reshape(B).astype(jnp.int32)
    hist_i_t = hist_item_id.astype(jnp.int32).T.reshape(H * B)
    hist_c_t = hist_cate_id.astype(jnp.int32).T.reshape(H * B)
    return _run(uid, ugen, iid, cid, hist_i_t, hist_c_t,
                emb_user_id, emb_user_gender, emb_item_id, emb_cate_id)
